# baseline (speedup 1.0000x reference)
import functools

import jax
import jax.numpy as jnp
from jax import lax
from jax.experimental import pallas as pl
from jax.experimental.pallas import tpu as pltpu
from jax.experimental.pallas import tpu_sc as plsc

F = 26
B = 16384
V = 100000
D = 64
NC = 2
NS = 16
FPC = F // NC
BPT = B // NS
OWN = 6144
CW = 512
NCH = OWN // CW
TAIL0 = 99968
TAILW = 32
XROW = 1032
NB = 128
STAGR = F * B + 16

_mesh = plsc.VectorSubcoreMesh(core_axis_name="c", subcore_axis_name="s")


@functools.partial(
    pl.kernel,
    out_type=(jax.ShapeDtypeStruct((STAGR, 128), jnp.float32),
              jax.ShapeDtypeStruct((NC * NS * NS * XROW,), jnp.int32)),
    mesh=_mesh,
    scratch_types=[
        pltpu.VMEM((BPT,), jnp.int32),
        pltpu.VMEM((NS * XROW,), jnp.int32),
        pltpu.VMEM((NS * XROW,), jnp.int32),
        pltpu.VMEM((BPT + 16,), jnp.int32),
        pltpu.VMEM((D, CW), jnp.float32),
        pltpu.VMEM((D, CW), jnp.float32),
        pltpu.VMEM((TAILW * D,), jnp.float32),
        pltpu.VMEM((NB, 128), jnp.float32),
        pltpu.VMEM((NB,), jnp.int32),
        pltpu.SMEM((NS,), jnp.int32),
        pltpu.SemaphoreType.DMA,
        pltpu.SemaphoreType.DMA,
        pltpu.SemaphoreType.DMA,
    ],
    compiler_params=pltpu.CompilerParams(
        use_tc_tiling_on_sc=True, needs_layout_passes=False),
)
def _extract(idx_hbm, tails_hbm, *rest):
    tbls = rest[:F]
    stag_hbm = rest[F]
    xch_hbm = rest[F + 1]
    (idx_v, obox, ibox, selb, slab0, slab1, tailv, rowb, idxb, cnt_sm,
     sem0, sem1, semf) = rest[F + 2:]

    cid = lax.axis_index("c")
    sid = lax.axis_index("s")
    fbase = cid * FPC
    iota = lax.iota(jnp.int32, 16)
    dumpv = F * B + iota

    for k in range(NB // 16):
        idxb[pl.ds(k * 16, 16)] = dumpv

    def flush():
        pltpu.async_copy(rowb, stag_hbm.at[idxb], semf).wait()
        for k in range(NB // 16):
            idxb[pl.ds(k * 16, 16)] = dumpv

    def fetch_async(f, slab, lo_c, sem):
        for i in range(F):
            @pl.when(f == i)
            def _():
                pltpu.async_copy(tbls[i].at[:, pl.ds(lo_c, CW)], slab, sem)

    def drain(slab, sem):
        pltpu.make_async_copy(tbls[0].at[:, pl.ds(0, CW)], slab, sem).wait()

    def process(f, slab, lo_c, width, is_tail, nrows0):
        def ext(nsel):
            def body(j, nrows):
                codes = selb[pl.ds(j * 16, 16)]
                lm = (j * 16 + iota) < nsel
                v = lax.shift_right_logical(codes, 14)
                b = codes & 16383
                c = jnp.where(lm, v - lo_c, 0)
                srow = jnp.where(lm, f * B + b, dumpv)
                local = nrows & (NB - 1)
                slotv = local + iota
                idxb[pl.ds(local, 16)] = srow
                if is_tail:
                    c64 = c * D
                    for d in range(D):
                        w = plsc.load_gather(tailv, [c64 + d])
                        plsc.store_scatter(
                            rowb, [slotv, jnp.full((16,), d, jnp.int32)], w)
                else:
                    for d in range(D):
                        dv = jnp.full((16,), d, jnp.int32)
                        w = plsc.load_gather(slab, [dv, c])
                        plsc.store_scatter(rowb, [slotv, dv], w)
                @pl.when(local == NB - 16)
                def _():
                    flush()
                return nrows + 16
            return body

        def src_body(src, nrows):
            cnt = ibox[pl.ds(src * XROW, 16)][0]
            def blk(j, ns):
                codes = ibox[pl.ds(src * XROW + 8 + j * 16, 16)]
                valid = (j * 16 + iota) < cnt
                v = lax.shift_right_logical(codes, 14)
                m = valid & (v >= lo_c) & (v < lo_c + width)
                plsc.store_compressed(selb.at[pl.ds(ns, 16)], codes, mask=m)
                return ns + plsc.all_reduce_population_count(m)[0]
            nsel = lax.fori_loop(0, (cnt + 15) // 16, blk, 0)
            return lax.fori_loop(0, (nsel + 15) // 16, ext(nsel), nrows)
        return lax.fori_loop(0, NS, src_body, nrows0)

    def table_body(ft, carry):
        f = fbase + ft
        lo_s = sid * OWN
        fetch_async(f, slab0, lo_s, sem0)
        pltpu.sync_copy(idx_hbm.at[pl.ds(f * B + sid * BPT, BPT)], idx_v)
        for o in range(NS):
            cnt_sm[o] = 0
        lane0 = iota == 0
        def binb(j, carry2):
            vecv = idx_v[pl.ds(j * 16, 16)]
            ovec = jnp.minimum(
                (lax.shift_right_logical(vecv, 11) * 43) >> 7, NS - 1)
            codev = vecv * B + (sid * BPT + j * 16 + iota)
            for k in range(16):
                o = ovec[k]
                cnt = cnt_sm[o]
                plsc.store_scatter(
                    obox, [jnp.full((16,), o * XROW + 8 + cnt, jnp.int32)],
                    jnp.full((16,), codev[k], jnp.int32), mask=lane0)
                cnt_sm[o] = cnt + 1
            return carry2
        lax.fori_loop(0, BPT // 16, binb, 0)
        for o in range(NS):
            plsc.store_scatter(
                obox, [jnp.full((16,), o * XROW, jnp.int32)],
                jnp.full((16,), cnt_sm[o], jnp.int32), mask=lane0)
        xbase = cid * (NS * NS * XROW)
        pltpu.sync_copy(
            obox, xch_hbm.at[pl.ds(xbase + sid * NS * XROW, NS * XROW)])
        plsc.subcore_barrier()
        for src in range(NS):
            pltpu.sync_copy(
                xch_hbm.at[pl.ds(xbase + src * NS * XROW + sid * XROW,
                                 XROW)],
                ibox.at[pl.ds(src * XROW, XROW)])
        plsc.subcore_barrier()

        nch = jnp.where(sid == NS - 1, 15, NCH)
        def c2body(c2, nrows):
            c_a = 2 * c2
            c_b = 2 * c2 + 1
            @pl.when(c_b < nch)
            def _():
                fetch_async(f, slab1, lo_s + c_b * CW, sem1)
            drain(slab0, sem0)
            nrows = process(f, slab0, lo_s + c_a * CW, CW, False, nrows)
            @pl.when(c_a + 2 < nch)
            def _():
                fetch_async(f, slab0, lo_s + (c_a + 2) * CW, sem0)
            def do_b(nr):
                drain(slab1, sem1)
                return process(f, slab1, lo_s + c_b * CW, CW, False, nr)
            nrows = lax.cond(c_b < nch, do_b, lambda nr: nr, nrows)
            return nrows
        nrows = lax.fori_loop(0, (nch + 1) // 2, c2body, carry)

        def s15(nr):
            for i in range(F):
                @pl.when(f == i)
                def _():
                    pltpu.sync_copy(tbls[i].at[:, pl.ds(99840, 128)],
                                    slab0.at[:, pl.ds(0, 128)])
            nr = process(f, slab0, 99840, 128, False, nr)
            pltpu.sync_copy(tails_hbm.at[pl.ds(f * TAILW * D, TAILW * D)],
                            tailv)
            return process(f, slab0, TAIL0, TAILW, True, nr)
        nrows = lax.cond(sid == NS - 1, s15, lambda nr: nr, nrows)
        return nrows

    nrows_f = lax.fori_loop(0, FPC, table_body, 0)
    @pl.when((nrows_f & (NB - 1)) != 0)
    def _():
        flush()


RPT = 256
NPASS = B // (NC * NS * RPT)


@functools.partial(
    pl.kernel,
    out_type=jax.ShapeDtypeStruct((B, 128), jnp.float32),
    mesh=_mesh,
    scratch_types=[
        pltpu.VMEM((RPT, 128), jnp.float32),
        pltpu.VMEM((RPT, 128), jnp.float32),
        pltpu.VMEM((RPT, 128), jnp.float32),
        pltpu.SemaphoreType.DMA,
        pltpu.SemaphoreType.DMA,
    ],
    compiler_params=pltpu.CompilerParams(
        use_tc_tiling_on_sc=True, needs_layout_passes=False),
)
def _reduce(stag_hbm, out_hbm, acc, tmp0, tmp1, sem0, sem1):
    cid = lax.axis_index("c")
    sid = lax.axis_index("s")
    wid = sid * NC + cid

    def accum(t):
        def body(i, carry):
            for k in range(D // 16):
                plsc.addupdate(acc.at[i, pl.ds(k * 16, 16)],
                               t[i, pl.ds(k * 16, 16)])
            return carry
        lax.fori_loop(0, RPT, body, 0)

    def pass_body(p, carry):
        base = p * (NC * NS * RPT) + wid * RPT
        pltpu.async_copy(stag_hbm.at[pl.ds(base, RPT)], acc, sem0).wait()
        pltpu.async_copy(stag_hbm.at[pl.ds(B + base, RPT)], tmp0, sem0)
        pltpu.async_copy(stag_hbm.at[pl.ds(2 * B + base, RPT)], tmp1, sem1)

        def fbody(fi, carry2):
            par = fi & 1
            @pl.when(par == 0)
            def _():
                pltpu.make_async_copy(
                    stag_hbm.at[pl.ds(0, RPT)], tmp0, sem0).wait()
                accum(tmp0)
                @pl.when(fi + 3 < F)
                def _():
                    pltpu.async_copy(
                        stag_hbm.at[pl.ds((fi + 3) * B + base, RPT)],
                        tmp0, sem0)
            @pl.when(par == 1)
            def _():
                pltpu.make_async_copy(
                    stag_hbm.at[pl.ds(0, RPT)], tmp1, sem1).wait()
                accum(tmp1)
                @pl.when(fi + 3 < F)
                def _():
                    pltpu.async_copy(
                        stag_hbm.at[pl.ds((fi + 3) * B + base, RPT)],
                        tmp1, sem1)
            return carry2

        lax.fori_loop(0, F - 1, fbody, 0)
        pltpu.sync_copy(acc, out_hbm.at[pl.ds(base, RPT)])
        return carry

    lax.fori_loop(0, NPASS, pass_body, 0)


def kernel(
    cat_0, cat_1, cat_2, cat_3, cat_4, cat_5, cat_6, cat_7, cat_8, cat_9,
    cat_10, cat_11, cat_12, cat_13, cat_14, cat_15, cat_16, cat_17, cat_18,
    cat_19, cat_20, cat_21, cat_22, cat_23, cat_24, cat_25,
    W_cat_0, W_cat_1, W_cat_2, W_cat_3, W_cat_4, W_cat_5, W_cat_6, W_cat_7,
    W_cat_8, W_cat_9, W_cat_10, W_cat_11, W_cat_12, W_cat_13, W_cat_14,
    W_cat_15, W_cat_16, W_cat_17, W_cat_18, W_cat_19, W_cat_20, W_cat_21,
    W_cat_22, W_cat_23, W_cat_24, W_cat_25,
):
    cats = [
        cat_0, cat_1, cat_2, cat_3, cat_4, cat_5, cat_6, cat_7, cat_8,
        cat_9, cat_10, cat_11, cat_12, cat_13, cat_14, cat_15, cat_16,
        cat_17, cat_18, cat_19, cat_20, cat_21, cat_22, cat_23, cat_24,
        cat_25,
    ]
    tables = [
        W_cat_0, W_cat_1, W_cat_2, W_cat_3, W_cat_4, W_cat_5, W_cat_6,
        W_cat_7, W_cat_8, W_cat_9, W_cat_10, W_cat_11, W_cat_12, W_cat_13,
        W_cat_14, W_cat_15, W_cat_16, W_cat_17, W_cat_18, W_cat_19,
        W_cat_20, W_cat_21, W_cat_22, W_cat_23, W_cat_24, W_cat_25,
    ]
    idx_flat = jnp.concatenate(cats)
    tails = jnp.stack([w[TAIL0:] for w in tables]).reshape(-1)
    staging, _ = _extract(idx_flat, tails, *[w.T for w in tables])
    return _reduce(staging)[:, :D]


# speedup vs baseline: 3.1731x; 3.1731x over previous
import functools

import jax
import jax.numpy as jnp
from jax import lax
from jax.experimental import pallas as pl
from jax.experimental.pallas import tpu as pltpu
from jax.experimental.pallas import tpu_sc as plsc

F = 26
B = 16384
D = 64
NC = 2
NS = 16
NW = NC * NS
BPW = B // NW
CHUNK = 128
NCH = BPW // CHUNK
LANES = 16
VECS = D // LANES

_mesh = plsc.VectorSubcoreMesh(core_axis_name="c", subcore_axis_name="s")


@functools.partial(
    pl.kernel,
    out_type=jax.ShapeDtypeStruct((B, D), jnp.float32),
    mesh=_mesh,
    scratch_types=[
        pltpu.VMEM((F * NCH, CHUNK), jnp.int32),
        pltpu.VMEM((BPW, D), jnp.float32),
        pltpu.VMEM((BPW, D), jnp.float32),
        pltpu.SemaphoreType.DMA,
    ],
    compiler_params=pltpu.CompilerParams(use_tc_tiling_on_sc=False),
)
def _encode(idx_hbm, *rest):
    tables = rest[:F]
    out_hbm = rest[F]
    idx_v, acc, tmp, sem = rest[F + 1:]

    wid = lax.axis_index("s") * NC + lax.axis_index("c")
    base = wid * BPW

    pltpu.sync_copy(idx_hbm.at[wid], idx_v)

    def gather_feature(f, dst):
        copies = []
        for c in range(NCH):
            copies.append(
                pltpu.async_copy(
                    tables[f].at[idx_v.at[f * NCH + c]],
                    dst.at[pl.ds(c * CHUNK, CHUNK)],
                    sem,
                )
            )
        for cp in copies:
            cp.wait()

    gather_feature(0, acc)

    def accumulate(_tmp, _acc):
        def body(i, carry):
            for v in range(VECS):
                plsc.addupdate(
                    _acc.at[i, pl.ds(v * LANES, LANES)],
                    _tmp[i, pl.ds(v * LANES, LANES)],
                )
            return carry
        lax.fori_loop(0, BPW, body, 0)

    for f in range(1, F):
        gather_feature(f, tmp)
        accumulate(tmp, acc)

    pltpu.sync_copy(acc, out_hbm.at[pl.ds(base, BPW)])


def kernel(
    cat_0, cat_1, cat_2, cat_3, cat_4, cat_5, cat_6, cat_7, cat_8, cat_9,
    cat_10, cat_11, cat_12, cat_13, cat_14, cat_15, cat_16, cat_17, cat_18,
    cat_19, cat_20, cat_21, cat_22, cat_23, cat_24, cat_25,
    W_cat_0, W_cat_1, W_cat_2, W_cat_3, W_cat_4, W_cat_5, W_cat_6, W_cat_7,
    W_cat_8, W_cat_9, W_cat_10, W_cat_11, W_cat_12, W_cat_13, W_cat_14,
    W_cat_15, W_cat_16, W_cat_17, W_cat_18, W_cat_19, W_cat_20, W_cat_21,
    W_cat_22, W_cat_23, W_cat_24, W_cat_25,
):
    cats = [
        cat_0, cat_1, cat_2, cat_3, cat_4, cat_5, cat_6, cat_7, cat_8,
        cat_9, cat_10, cat_11, cat_12, cat_13, cat_14, cat_15, cat_16,
        cat_17, cat_18, cat_19, cat_20, cat_21, cat_22, cat_23, cat_24,
        cat_25,
    ]
    tables = [
        W_cat_0, W_cat_1, W_cat_2, W_cat_3, W_cat_4, W_cat_5, W_cat_6,
        W_cat_7, W_cat_8, W_cat_9, W_cat_10, W_cat_11, W_cat_12, W_cat_13,
        W_cat_14, W_cat_15, W_cat_16, W_cat_17, W_cat_18, W_cat_19,
        W_cat_20, W_cat_21, W_cat_22, W_cat_23, W_cat_24, W_cat_25,
    ]
    idx = jnp.stack(cats)
    idx = idx.reshape(F, NW, BPW).transpose(1, 0, 2)
    idx = idx.reshape(NW, F * NCH, CHUNK)
    return _encode(idx, *tables)
